# baseline (speedup 1.0000x reference)
import functools

import jax
import jax.numpy as jnp
from jax import lax
from jax.experimental import pallas as pl
from jax.experimental.pallas import tpu as pltpu

N = 50000
E = 800000

EDGE_BLK = 8000
NODE_BLK = 2000


def _dot(a, b):
    return jax.lax.dot_general(a, b, (((1,), (0,)), ((), ())),
                               preferred_element_type=jnp.float32)


def _edge_body(xg_ref, ea_ref, w1p_ref, w9_ref, b1_ref, w2_ref, b2_ref,
               wh1p_ref, whe_ref, bh1_ref, wh2_ref, bh2_ref,
               e_ref, m_ref):
    xg = xg_ref[...]
    ea = ea_ref[...]
    e1 = jnp.maximum(_dot(xg, w1p_ref[...]) + ea * w9_ref[...] + b1_ref[...],
                     0.0)
    e = jnp.maximum(_dot(e1, w2_ref[...]) + b2_ref[...], 0.0)
    e_ref[...] = e
    m1 = jnp.maximum(_dot(xg, wh1p_ref[...]) + _dot(e, whe_ref[...])
                     + bh1_ref[...], 0.0)
    m_ref[...] = jnp.maximum(_dot(m1, wh2_ref[...]) + bh2_ref[...], 0.0)


def _edge_mlps(xg, edge_attr, w1p, w9, b1, w2, b2, wh1p, whe, bh1, wh2, bh2):
    nblk = E // EDGE_BLK
    full = lambda shape: pl.BlockSpec(shape, lambda i: (0,) * len(shape))
    return pl.pallas_call(
        _edge_body,
        grid=(nblk,),
        in_specs=[
            pl.BlockSpec((EDGE_BLK, 32), lambda i: (i, 0)),
            pl.BlockSpec((EDGE_BLK, 1), lambda i: (i, 0)),
            full((32, 84)), full((1, 84)), full((1, 84)),
            full((84, 39)), full((1, 39)),
            full((32, 64)), full((39, 64)), full((1, 64)),
            full((64, 128)), full((1, 128)),
        ],
        out_specs=[
            pl.BlockSpec((EDGE_BLK, 39), lambda i: (i, 0)),
            pl.BlockSpec((EDGE_BLK, 128), lambda i: (i, 0)),
        ],
        out_shape=[
            jax.ShapeDtypeStruct((E, 39), jnp.float32),
            jax.ShapeDtypeStruct((E, 128), jnp.float32),
        ],
    )(xg, edge_attr, w1p, w9, b1, w2, b2, wh1p, whe, bh1, wh2, bh2)


def _node_body(x_ref, s_ref, cnt_ref, wp1_ref, wm1_ref, b1_ref, w2_ref, b2_ref,
               xo_ref):
    s = s_ref[...]
    cnt = jnp.maximum(cnt_ref[...], 1.0)
    mean = s / cnt
    u1 = jnp.maximum(_dot(x_ref[...], wp1_ref[...]) + _dot(mean, wm1_ref[...])
                     + b1_ref[...], 0.0)
    xo_ref[...] = jnp.maximum(_dot(u1, w2_ref[...]) + b2_ref[...], 0.0)


def _node_mlp(x, s, cnt, wp1, wm1, b1, w2, b2):
    nblk = N // NODE_BLK
    full = lambda shape: pl.BlockSpec(shape, lambda i: (0,) * len(shape))
    return pl.pallas_call(
        _node_body,
        grid=(nblk,),
        in_specs=[
            pl.BlockSpec((NODE_BLK, 4), lambda i: (i, 0)),
            pl.BlockSpec((NODE_BLK, 128), lambda i: (i, 0)),
            pl.BlockSpec((NODE_BLK, 1), lambda i: (i, 0)),
            full((4, 256)), full((128, 256)), full((1, 256)),
            full((256, 256)), full((1, 256)),
        ],
        out_specs=pl.BlockSpec((NODE_BLK, 256), lambda i: (i, 0)),
        out_shape=jax.ShapeDtypeStruct((N, 256), jnp.float32),
    )(x, s, cnt, wp1, wm1, b1, w2, b2)


def kernel(x, edge_index, edge_attr, W_e1, b_e1, W_e2, b_e2,
           Wn1_1, bn1_1, Wn1_2, bn1_2, Wn2_1, bn2_1, Wn2_2, bn2_2):
    row = edge_index[0]

    x16 = jnp.pad(x, ((0, 0), (0, 12)))
    idx = jnp.reshape(jnp.transpose(edge_index), (2 * E,))
    xg = jnp.reshape(x16[idx], (E, 32))

    w1p = jnp.zeros((32, 84), jnp.float32).at[0:4].set(W_e1[0:4]).at[16:20].set(W_e1[4:8])
    w9 = W_e1[8:9]
    wh1p = jnp.zeros((32, 64), jnp.float32).at[16:20].set(Wn1_1[0:4])
    whe = Wn1_1[4:43]
    wp1 = Wn2_1[0:4]
    wm1 = Wn2_1[4:132]

    e, m = _edge_mlps(xg, edge_attr, w1p, w9, b_e1[None], W_e2, b_e2[None],
                      wh1p, whe, bn1_1[None], Wn1_2, bn1_2[None])

    s = jax.ops.segment_sum(m, row, num_segments=N)
    cnt = jax.ops.segment_sum(jnp.ones((E, 1), jnp.float32), row, num_segments=N)

    xo = _node_mlp(x, s, cnt, wp1, wm1, bn2_1[None], Wn2_2, bn2_2[None])
    return (xo, e)


# speedup vs baseline: 3.1582x; 3.1582x over previous
import functools

import jax
import jax.numpy as jnp
from jax import lax
from jax.experimental import pallas as pl
from jax.experimental.pallas import tpu as pltpu
from jax.experimental.pallas import tpu_sc as plsc

N = 50000
E = 800000

NC = 2
NS = 16

EDGE_BLK = 8000
NODE_BLK = 2000

GK = 2000
GSUB = 125
G_NSUB = (2 * E) // (GK * NC * NS)

_sc_mesh = plsc.VectorSubcoreMesh(core_axis_name="c", subcore_axis_name="s")


def _gather_body(x16_ref, idx2d_ref, out_ref, idxv, rows):
    c = lax.axis_index("c")
    s = lax.axis_index("s")
    wid = s * NC + c

    def jloop(j, carry):
        t = wid + (NC * NS) * j
        pltpu.sync_copy(idx2d_ref.at[pl.ds(t * (GK // GSUB), GK // GSUB)], idxv)
        for jj in range(GK // GSUB):
            pltpu.sync_copy(x16_ref.at[idxv.at[jj]],
                            rows.at[pl.ds(jj * GSUB, GSUB)])
        pltpu.sync_copy(rows, out_ref.at[pl.ds(t * GK, GK)])
        return carry

    lax.fori_loop(0, G_NSUB, jloop, 0)


@functools.partial(jax.jit, static_argnames=())
def _sc_gather(x16, idx2d):
    return pl.kernel(
        _gather_body,
        out_type=jax.ShapeDtypeStruct((2 * E, 16), jnp.float32),
        mesh=_sc_mesh,
        scratch_types=[
            pltpu.VMEM((GK // GSUB, GSUB), jnp.int32),
            pltpu.VMEM((GK, 16), jnp.float32),
        ],
        compiler_params=pltpu.CompilerParams(use_tc_tiling_on_sc=False),
    )(x16, idx2d)


SK = 500
S_NSUB = E // (SK * NS)
SK2 = 500
C_NSUB = E // (SK2 * NC * NS)
ROWS_PT = N // NS
WCH = 125


def _scatter_body(m_ref, row2d_ref, zeros_ref, ones_ref,
                  s_ref, cnta_ref, cntb_ref,
                  acc, idxv, mrows, zbuf, wbuf, obuf):
    c = lax.axis_index("c")
    s_id = lax.axis_index("s")
    wid = s_id * NC + c
    rbase = s_id * ROWS_PT

    pltpu.sync_copy(zeros_ref, zbuf)
    pltpu.sync_copy(ones_ref, obuf)
    for k in range(ROWS_PT // WCH):
        pltpu.sync_copy(zbuf, acc.at[pl.ds(rbase + k * WCH, WCH)])
    plsc.subcore_barrier()

    for p in range(2):
        coff = (2 * c + p) * 32

        def jloop(j, carry):
            t = s_id + NS * j
            pltpu.sync_copy(row2d_ref.at[pl.ds(t * (SK // GSUB), SK // GSUB)],
                            idxv)
            pltpu.sync_copy(m_ref.at[pl.ds(t * SK, SK), pl.ds(coff, 32)],
                            mrows)
            for jj in range(SK // GSUB):
                pltpu.sync_copy(mrows.at[pl.ds(jj * GSUB, GSUB)],
                                acc.at[idxv.at[jj]], add=True)
            return carry

        lax.fori_loop(0, S_NSUB, jloop, 0)
        plsc.subcore_barrier()
        for k in range(ROWS_PT // WCH):
            rb = rbase + k * WCH
            pltpu.sync_copy(acc.at[pl.ds(rb, WCH)], wbuf)
            pltpu.sync_copy(wbuf, s_ref.at[pl.ds(rb, WCH), pl.ds(coff, 32)])
            pltpu.sync_copy(zbuf, acc.at[pl.ds(rb, WCH)])
        plsc.subcore_barrier()

    def jloop2(j, carry):
        t2 = wid + NC * NS * j
        pltpu.sync_copy(row2d_ref.at[pl.ds(t2 * (SK2 // GSUB), SK2 // GSUB)],
                        idxv)
        for jj in range(SK2 // GSUB):
            pltpu.sync_copy(obuf, acc.at[idxv.at[jj]], add=True)
        return carry

    lax.fori_loop(0, C_NSUB, jloop2, 0)
    plsc.subcore_barrier()
    for k in range(ROWS_PT // WCH):
        rb = rbase + k * WCH
        pltpu.sync_copy(acc.at[pl.ds(rb, WCH)], wbuf)

        @pl.when(c == 0)
        def _():
            pltpu.sync_copy(wbuf, cnta_ref.at[pl.ds(rb, WCH)])

        @pl.when(c == 1)
        def _():
            pltpu.sync_copy(wbuf, cntb_ref.at[pl.ds(rb, WCH)])


def _sc_scatter(m, row2d, zeros, ones):
    return pl.kernel(
        _scatter_body,
        out_type=[
            jax.ShapeDtypeStruct((N, 128), jnp.float32),
            jax.ShapeDtypeStruct((N, 32), jnp.float32),
            jax.ShapeDtypeStruct((N, 32), jnp.float32),
        ],
        mesh=_sc_mesh,
        scratch_types=[
            pltpu.VMEM_SHARED((N, 32), jnp.float32),
            pltpu.VMEM((SK2 // GSUB, GSUB), jnp.int32),
            pltpu.VMEM((SK, 32), jnp.float32),
            pltpu.VMEM((WCH, 32), jnp.float32),
            pltpu.VMEM((WCH, 32), jnp.float32),
            pltpu.VMEM((GSUB, 32), jnp.float32),
        ],
        compiler_params=pltpu.CompilerParams(use_tc_tiling_on_sc=False),
    )(m, row2d, zeros, ones)


def _dot(a, b):
    return jax.lax.dot_general(a, b, (((1,), (0,)), ((), ())),
                               preferred_element_type=jnp.float32)


def _edge_body(xg_ref, ea_ref, w1p_ref, w9_ref, b1_ref, w2_ref, b2_ref,
               wh1p_ref, whe_ref, bh1_ref, wh2_ref, bh2_ref,
               e_ref, m_ref):
    xg = xg_ref[...]
    ea = ea_ref[...]
    e1 = jnp.maximum(_dot(xg, w1p_ref[...]) + ea * w9_ref[...] + b1_ref[...],
                     0.0)
    e = jnp.maximum(_dot(e1, w2_ref[...]) + b2_ref[...], 0.0)
    e_ref[...] = e
    m1 = jnp.maximum(_dot(xg, wh1p_ref[...]) + _dot(e, whe_ref[...])
                     + bh1_ref[...], 0.0)
    m_ref[...] = jnp.maximum(_dot(m1, wh2_ref[...]) + bh2_ref[...], 0.0)


def _edge_mlps(xg, edge_attr, w1p, w9, b1, w2, b2, wh1p, whe, bh1, wh2, bh2):
    nblk = E // EDGE_BLK
    full = lambda shape: pl.BlockSpec(shape, lambda i: (0,) * len(shape))
    return pl.pallas_call(
        _edge_body,
        grid=(nblk,),
        in_specs=[
            pl.BlockSpec((EDGE_BLK, 32), lambda i: (i, 0)),
            pl.BlockSpec((EDGE_BLK, 1), lambda i: (i, 0)),
            full((32, 84)), full((1, 84)), full((1, 84)),
            full((84, 39)), full((1, 39)),
            full((32, 64)), full((39, 64)), full((1, 64)),
            full((64, 128)), full((1, 128)),
        ],
        out_specs=[
            pl.BlockSpec((EDGE_BLK, 39), lambda i: (i, 0)),
            pl.BlockSpec((EDGE_BLK, 128), lambda i: (i, 0)),
        ],
        out_shape=[
            jax.ShapeDtypeStruct((E, 39), jnp.float32),
            jax.ShapeDtypeStruct((E, 128), jnp.float32),
        ],
    )(xg, edge_attr, w1p, w9, b1, w2, b2, wh1p, whe, bh1, wh2, bh2)


def _node_body(x_ref, s_ref, cnta_ref, cntb_ref, wp1_ref, wm1_ref, b1_ref,
               w2_ref, b2_ref, xo_ref):
    s = s_ref[...]
    cnt = jnp.maximum(cnta_ref[...][:, 0:1] + cntb_ref[...][:, 0:1], 1.0)
    mean = s / cnt
    u1 = jnp.maximum(_dot(x_ref[...], wp1_ref[...]) + _dot(mean, wm1_ref[...])
                     + b1_ref[...], 0.0)
    xo_ref[...] = jnp.maximum(_dot(u1, w2_ref[...]) + b2_ref[...], 0.0)


def _node_mlp(x, s, cnta, cntb, wp1, wm1, b1, w2, b2):
    nblk = N // NODE_BLK
    full = lambda shape: pl.BlockSpec(shape, lambda i: (0,) * len(shape))
    return pl.pallas_call(
        _node_body,
        grid=(nblk,),
        in_specs=[
            pl.BlockSpec((NODE_BLK, 4), lambda i: (i, 0)),
            pl.BlockSpec((NODE_BLK, 128), lambda i: (i, 0)),
            pl.BlockSpec((NODE_BLK, 32), lambda i: (i, 0)),
            pl.BlockSpec((NODE_BLK, 32), lambda i: (i, 0)),
            full((4, 256)), full((128, 256)), full((1, 256)),
            full((256, 256)), full((1, 256)),
        ],
        out_specs=pl.BlockSpec((NODE_BLK, 256), lambda i: (i, 0)),
        out_shape=jax.ShapeDtypeStruct((N, 256), jnp.float32),
    )(x, s, cnta, cntb, wp1, wm1, b1, w2, b2)


def kernel(x, edge_index, edge_attr, W_e1, b_e1, W_e2, b_e2,
           Wn1_1, bn1_1, Wn1_2, bn1_2, Wn2_1, bn2_1, Wn2_2, bn2_2):
    x16 = jnp.pad(x, ((0, 0), (0, 12)))
    idx2d = jnp.reshape(jnp.transpose(edge_index), (2 * E // GSUB, GSUB))
    xg = jnp.reshape(_sc_gather(x16, idx2d), (E, 32))

    w1p = jnp.zeros((32, 84), jnp.float32).at[0:4].set(W_e1[0:4]).at[16:20].set(W_e1[4:8])
    w9 = W_e1[8:9]
    wh1p = jnp.zeros((32, 64), jnp.float32).at[16:20].set(Wn1_1[0:4])
    whe = Wn1_1[4:43]
    wp1 = Wn2_1[0:4]
    wm1 = Wn2_1[4:132]

    e, m = _edge_mlps(xg, edge_attr, w1p, w9, b_e1[None], W_e2, b_e2[None],
                      wh1p, whe, bn1_1[None], Wn1_2, bn1_2[None])

    row2d = jnp.reshape(edge_index[0], (E // GSUB, GSUB))
    zeros = jnp.zeros((WCH, 32), jnp.float32)
    ones = jnp.ones((GSUB, 32), jnp.float32)
    s, cnta, cntb = _sc_scatter(m, row2d, zeros, ones)

    xo = _node_mlp(x, s, cnta, cntb, wp1, wm1, bn2_1[None], Wn2_2, bn2_2[None])
    return (xo, e)


# speedup vs baseline: 3.1616x; 1.0011x over previous
import functools

import jax
import jax.numpy as jnp
from jax import lax
from jax.experimental import pallas as pl
from jax.experimental.pallas import tpu as pltpu
from jax.experimental.pallas import tpu_sc as plsc

N = 50000
E = 800000

NC = 2
NS = 16

EDGE_BLK = 8000
NODE_BLK = 2000

GK = 2000
GSUB = 125
G_NSUB = (2 * E) // (GK * NC * NS)

_sc_mesh = plsc.VectorSubcoreMesh(core_axis_name="c", subcore_axis_name="s")


def _gather_body(x16_ref, idx2d_ref, out_ref, idxv, rows):
    c = lax.axis_index("c")
    s = lax.axis_index("s")
    wid = s * NC + c

    def jloop(j, carry):
        t = wid + (NC * NS) * j
        pltpu.sync_copy(idx2d_ref.at[pl.ds(t * (GK // GSUB), GK // GSUB)], idxv)
        for jj in range(GK // GSUB):
            pltpu.sync_copy(x16_ref.at[idxv.at[jj]],
                            rows.at[pl.ds(jj * GSUB, GSUB)])
        pltpu.sync_copy(rows, out_ref.at[pl.ds(t * GK, GK)])
        return carry

    lax.fori_loop(0, G_NSUB, jloop, 0)


@functools.partial(jax.jit, static_argnames=())
def _sc_gather(x16, idx2d):
    return pl.kernel(
        _gather_body,
        out_type=jax.ShapeDtypeStruct((2 * E, 16), jnp.float32),
        mesh=_sc_mesh,
        scratch_types=[
            pltpu.VMEM((GK // GSUB, GSUB), jnp.int32),
            pltpu.VMEM((GK, 16), jnp.float32),
        ],
        compiler_params=pltpu.CompilerParams(use_tc_tiling_on_sc=False),
    )(x16, idx2d)


SK = 500
S_NSUB = E // (SK * NS)
SK2 = 500
C_NSUB = E // (SK2 * NC * NS)
ROWS_PT = N // NS
WCH = 125


def _scatter_body(m_ref, row2d_ref, zeros_ref, ones_ref,
                  s_ref, cnta_ref, cntb_ref,
                  acc, idxv, mrows, zbuf, wbuf, obuf):
    c = lax.axis_index("c")
    s_id = lax.axis_index("s")
    wid = s_id * NC + c
    rbase = s_id * ROWS_PT

    pltpu.sync_copy(zeros_ref, zbuf)
    pltpu.sync_copy(ones_ref, obuf)
    for k in range(ROWS_PT // WCH):
        pltpu.sync_copy(zbuf, acc.at[pl.ds(rbase + k * WCH, WCH)])
    plsc.subcore_barrier()

    for p in range(2):
        coff = (2 * c + p) * 32

        def jloop(j, carry):
            t = s_id + NS * j
            pltpu.sync_copy(row2d_ref.at[pl.ds(t * (SK // GSUB), SK // GSUB)],
                            idxv)
            pltpu.sync_copy(m_ref.at[pl.ds(t * SK, SK), pl.ds(coff, 32)],
                            mrows)
            for jj in range(SK // GSUB):
                pltpu.sync_copy(mrows.at[pl.ds(jj * GSUB, GSUB)],
                                acc.at[idxv.at[jj]], add=True)
            return carry

        lax.fori_loop(0, S_NSUB, jloop, 0)
        plsc.subcore_barrier()
        for k in range(ROWS_PT // WCH):
            rb = rbase + k * WCH
            pltpu.sync_copy(acc.at[pl.ds(rb, WCH)], wbuf)
            pltpu.sync_copy(wbuf, s_ref.at[pl.ds(rb, WCH), pl.ds(coff, 32)])
            pltpu.sync_copy(zbuf, acc.at[pl.ds(rb, WCH)])
        plsc.subcore_barrier()

    def jloop2(j, carry):
        t2 = wid + NC * NS * j
        pltpu.sync_copy(row2d_ref.at[pl.ds(t2 * (SK2 // GSUB), SK2 // GSUB)],
                        idxv)
        for jj in range(SK2 // GSUB):
            pltpu.sync_copy(obuf, acc.at[idxv.at[jj]], add=True)
        return carry

    lax.fori_loop(0, C_NSUB, jloop2, 0)
    plsc.subcore_barrier()
    for k in range(ROWS_PT // WCH):
        rb = rbase + k * WCH
        pltpu.sync_copy(acc.at[pl.ds(rb, WCH)], wbuf)

        @pl.when(c == 0)
        def _():
            pltpu.sync_copy(wbuf, cnta_ref.at[pl.ds(rb, WCH)])

        @pl.when(c == 1)
        def _():
            pltpu.sync_copy(wbuf, cntb_ref.at[pl.ds(rb, WCH)])


def _sc_scatter(m, row2d, zeros, ones):
    return pl.kernel(
        _scatter_body,
        out_type=[
            jax.ShapeDtypeStruct((N, 128), jnp.float32),
            jax.ShapeDtypeStruct((N, 32), jnp.float32),
            jax.ShapeDtypeStruct((N, 32), jnp.float32),
        ],
        mesh=_sc_mesh,
        scratch_types=[
            pltpu.VMEM_SHARED((N, 32), jnp.float32),
            pltpu.VMEM((SK2 // GSUB, GSUB), jnp.int32),
            pltpu.VMEM((SK, 32), jnp.float32),
            pltpu.VMEM((WCH, 32), jnp.float32),
            pltpu.VMEM((WCH, 32), jnp.float32),
            pltpu.VMEM((GSUB, 32), jnp.float32),
        ],
        compiler_params=pltpu.CompilerParams(use_tc_tiling_on_sc=False),
    )(m, row2d, zeros, ones)


def _dot(a, b):
    return jax.lax.dot_general(a, b, (((1,), (0,)), ((), ())),
                               preferred_element_type=jnp.float32)


def _bf(v):
    return v.astype(jnp.bfloat16)


def _edge_body(xg_ref, ea_ref, w1p_ref, w9_ref, b1_ref, w2_ref, b2_ref,
               wh1p_ref, whe_ref, bh1_ref, wh2_ref, bh2_ref,
               e_ref, m_ref):
    xg = _bf(xg_ref[...])
    ea = ea_ref[...]
    e1 = jnp.maximum(_dot(xg, w1p_ref[...]) + ea * w9_ref[...] + b1_ref[...],
                     0.0)
    e = jnp.maximum(_dot(_bf(e1), w2_ref[...]) + b2_ref[...], 0.0)
    e_ref[...] = e
    eb = _bf(e)
    m1 = jnp.maximum(_dot(xg, wh1p_ref[...]) + _dot(eb, whe_ref[...])
                     + bh1_ref[...], 0.0)
    m = jnp.maximum(_dot(_bf(m1), wh2_ref[...]) + bh2_ref[...], 0.0)
    m_ref[...] = jnp.reshape(m, (EDGE_BLK * 128,))


def _edge_mlps(xg, edge_attr, w1p, w9, b1, w2, b2, wh1p, whe, bh1, wh2, bh2):
    nblk = E // EDGE_BLK
    full = lambda shape: pl.BlockSpec(shape, lambda i: (0,) * len(shape))
    return pl.pallas_call(
        _edge_body,
        grid=(nblk,),
        in_specs=[
            pl.BlockSpec((EDGE_BLK, 32), lambda i: (i, 0)),
            pl.BlockSpec((EDGE_BLK, 1), lambda i: (i, 0)),
            full((32, 84)), full((1, 84)), full((1, 84)),
            full((84, 39)), full((1, 39)),
            full((32, 64)), full((39, 64)), full((1, 64)),
            full((64, 128)), full((1, 128)),
        ],
        out_specs=[
            pl.BlockSpec((EDGE_BLK, 39), lambda i: (i, 0)),
            pl.BlockSpec((EDGE_BLK * 128,), lambda i: (i,)),
        ],
        out_shape=[
            jax.ShapeDtypeStruct((E, 39), jnp.float32),
            jax.ShapeDtypeStruct((E * 128,), jnp.float32),
        ],
    )(xg, edge_attr, w1p, w9, b1, w2, b2, wh1p, whe, bh1, wh2, bh2)


def _node_body(x_ref, s_ref, cnta_ref, cntb_ref, wp1_ref, wm1_ref, b1_ref,
               w2_ref, b2_ref, xo_ref):
    s = s_ref[...]
    cnt = jnp.maximum(cnta_ref[...][:, 0:1] + cntb_ref[...][:, 0:1], 1.0)
    mean = _bf(s / cnt)
    u1 = jnp.maximum(_dot(_bf(x_ref[...]), wp1_ref[...])
                     + _dot(mean, wm1_ref[...]) + b1_ref[...], 0.0)
    xo_ref[...] = jnp.maximum(_dot(_bf(u1), w2_ref[...]) + b2_ref[...], 0.0)


def _node_mlp(x, s, cnta, cntb, wp1, wm1, b1, w2, b2):
    nblk = N // NODE_BLK
    full = lambda shape: pl.BlockSpec(shape, lambda i: (0,) * len(shape))
    return pl.pallas_call(
        _node_body,
        grid=(nblk,),
        in_specs=[
            pl.BlockSpec((NODE_BLK, 4), lambda i: (i, 0)),
            pl.BlockSpec((NODE_BLK, 128), lambda i: (i, 0)),
            pl.BlockSpec((NODE_BLK, 32), lambda i: (i, 0)),
            pl.BlockSpec((NODE_BLK, 32), lambda i: (i, 0)),
            full((4, 256)), full((128, 256)), full((1, 256)),
            full((256, 256)), full((1, 256)),
        ],
        out_specs=pl.BlockSpec((NODE_BLK, 256), lambda i: (i, 0)),
        out_shape=jax.ShapeDtypeStruct((N, 256), jnp.float32),
    )(x, s, cnta, cntb, wp1, wm1, b1, w2, b2)


def kernel(x, edge_index, edge_attr, W_e1, b_e1, W_e2, b_e2,
           Wn1_1, bn1_1, Wn1_2, bn1_2, Wn2_1, bn2_1, Wn2_2, bn2_2):
    x16 = jnp.pad(x, ((0, 0), (0, 12)))
    idx2d = jnp.reshape(jnp.transpose(edge_index), (2 * E // GSUB, GSUB))
    xg = jnp.reshape(_sc_gather(x16, idx2d), (E, 32))

    bf = jnp.bfloat16
    w1p = (jnp.zeros((32, 84), jnp.float32).at[0:4].set(W_e1[0:4])
           .at[16:20].set(W_e1[4:8])).astype(bf)
    w9 = W_e1[8:9]
    wh1p = jnp.zeros((32, 64), jnp.float32).at[16:20].set(Wn1_1[0:4]).astype(bf)
    whe = Wn1_1[4:43].astype(bf)
    wp1 = Wn2_1[0:4].astype(bf)
    wm1 = Wn2_1[4:132].astype(bf)

    e, mflat = _edge_mlps(xg, edge_attr, w1p, w9, b_e1[None],
                          W_e2.astype(bf), b_e2[None], wh1p, whe, bn1_1[None],
                          Wn1_2.astype(bf), bn1_2[None])
    m = jnp.reshape(mflat, (E, 128))

    row2d = jnp.reshape(edge_index[0], (E // GSUB, GSUB))
    zeros = jnp.zeros((WCH, 32), jnp.float32)
    ones = jnp.ones((GSUB, 32), jnp.float32)
    s, cnta, cntb = _sc_scatter(m, row2d, zeros, ones)

    xo = _node_mlp(x, s, cnta, cntb, wp1, wm1, bn2_1[None],
                   Wn2_2.astype(bf), bn2_2[None])
    return (xo, e)


# speedup vs baseline: 5.1185x; 1.6190x over previous
import functools

import jax
import jax.numpy as jnp
from jax import lax
from jax.experimental import pallas as pl
from jax.experimental.pallas import tpu as pltpu
from jax.experimental.pallas import tpu_sc as plsc

N = 50000
E = 800000

NC = 2
NS = 16

EDGE_BLK = 6400
NODE_BLK = 2000

GK = 2000
GSUB = 125
G_NSUB = (2 * E) // (GK * NC * NS)

_sc_mesh = plsc.VectorSubcoreMesh(core_axis_name="c", subcore_axis_name="s")


GKE = 1000
NSEC = E // GK


def _gather_body(x16_ref, idxcat2d_ref, out_ref, idxv, xsv):
    c = lax.axis_index("c")
    s = lax.axis_index("s")
    wid = s * NC + c
    nsub = GK // GSUB

    def jloop(j, carry):
        t = wid + (NC * NS) * j
        ebase = (t % NSEC) * GK
        colo = (t // NSEC) * 16
        pltpu.sync_copy(idxcat2d_ref.at[pl.ds(t * nsub, nsub)], idxv)
        for jj in range(nsub):
            pltpu.sync_copy(x16_ref.at[idxv.at[jj]],
                            xsv.at[pl.ds(jj * GSUB, GSUB)])
        pltpu.sync_copy(xsv, out_ref.at[pl.ds(ebase, GK), pl.ds(colo, 16)])
        return carry

    lax.fori_loop(0, G_NSUB, jloop, 0)


def _sc_gather(x16, idxcat2d):
    return pl.kernel(
        _gather_body,
        out_type=jax.ShapeDtypeStruct((E, 128), jnp.float32),
        mesh=_sc_mesh,
        scratch_types=[
            pltpu.VMEM((GK // GSUB, GSUB), jnp.int32),
            pltpu.VMEM((GK, 16), jnp.float32),
        ],
        compiler_params=pltpu.CompilerParams(use_tc_tiling_on_sc=False),
    )(x16, idxcat2d)


SK = 500
S_NSUB = E // (SK * NS)
SK2 = 500
C_NSUB = E // (SK2 * NC * NS)
ROWS_PT = N // NS
WCH = 125


def _scatter_body(m_ref, row2d_ref, zeros_ref, ones_ref,
                  s_ref, cnta_ref, cntb_ref,
                  acc, idxv, mrows, zbuf, wbuf, obuf):
    c = lax.axis_index("c")
    s_id = lax.axis_index("s")
    wid = s_id * NC + c
    rbase = s_id * ROWS_PT

    pltpu.sync_copy(zeros_ref, zbuf)
    pltpu.sync_copy(ones_ref, obuf)
    for k in range(ROWS_PT // WCH):
        pltpu.sync_copy(zbuf, acc.at[pl.ds(rbase + k * WCH, WCH)])
    plsc.subcore_barrier()

    for p in range(2):
        coff = (2 * c + p) * 32

        def jloop(j, carry):
            t = s_id + NS * j
            pltpu.sync_copy(row2d_ref.at[pl.ds(t * (SK // GSUB), SK // GSUB)],
                            idxv)
            pltpu.sync_copy(m_ref.at[pl.ds(t * SK, SK), pl.ds(coff, 32)],
                            mrows)
            for jj in range(SK // GSUB):
                pltpu.sync_copy(mrows.at[pl.ds(jj * GSUB, GSUB)],
                                acc.at[idxv.at[jj]], add=True)
            return carry

        lax.fori_loop(0, S_NSUB, jloop, 0)
        plsc.subcore_barrier()
        for k in range(ROWS_PT // WCH):
            rb = rbase + k * WCH
            pltpu.sync_copy(acc.at[pl.ds(rb, WCH)], wbuf)
            pltpu.sync_copy(wbuf, s_ref.at[pl.ds(rb, WCH), pl.ds(coff, 32)])
            pltpu.sync_copy(zbuf, acc.at[pl.ds(rb, WCH)])
        plsc.subcore_barrier()

    def jloop2(j, carry):
        t2 = wid + NC * NS * j
        pltpu.sync_copy(row2d_ref.at[pl.ds(t2 * (SK2 // GSUB), SK2 // GSUB)],
                        idxv)
        for jj in range(SK2 // GSUB):
            pltpu.sync_copy(obuf, acc.at[idxv.at[jj]], add=True)
        return carry

    lax.fori_loop(0, C_NSUB, jloop2, 0)
    plsc.subcore_barrier()
    for k in range(ROWS_PT // WCH):
        rb = rbase + k * WCH
        pltpu.sync_copy(acc.at[pl.ds(rb, WCH)], wbuf)

        @pl.when(c == 0)
        def _():
            pltpu.sync_copy(wbuf, cnta_ref.at[pl.ds(rb, WCH)])

        @pl.when(c == 1)
        def _():
            pltpu.sync_copy(wbuf, cntb_ref.at[pl.ds(rb, WCH)])


def _sc_scatter(m, row2d, zeros, ones):
    return pl.kernel(
        _scatter_body,
        out_type=[
            jax.ShapeDtypeStruct((N, 128), jnp.float32),
            jax.ShapeDtypeStruct((N, 32), jnp.float32),
            jax.ShapeDtypeStruct((N, 32), jnp.float32),
        ],
        mesh=_sc_mesh,
        scratch_types=[
            pltpu.VMEM_SHARED((N, 32), jnp.float32),
            pltpu.VMEM((SK2 // GSUB, GSUB), jnp.int32),
            pltpu.VMEM((SK, 32), jnp.float32),
            pltpu.VMEM((WCH, 32), jnp.float32),
            pltpu.VMEM((WCH, 32), jnp.float32),
            pltpu.VMEM((GSUB, 32), jnp.float32),
        ],
        compiler_params=pltpu.CompilerParams(use_tc_tiling_on_sc=False),
    )(m, row2d, zeros, ones)


def _dot(a, b):
    return jax.lax.dot_general(a, b, (((1,), (0,)), ((), ())),
                               preferred_element_type=jnp.float32)


def _bf(v):
    return v.astype(jnp.bfloat16)


def _edge_body(xg_ref, ea_ref, w1p_ref, w9_ref, b1_ref, w2_ref, b2_ref,
               wh1p_ref, whe_ref, bh1_ref, wh2_ref, bh2_ref,
               e_ref, m_ref):
    xg = _bf(xg_ref[...][:, 0:32])
    ea = ea_ref[...]
    e1 = jnp.maximum(_dot(xg, w1p_ref[...]) + ea * w9_ref[...] + b1_ref[...],
                     0.0)
    e = jnp.maximum(_dot(_bf(e1), w2_ref[...]) + b2_ref[...], 0.0)
    e_ref[...] = e
    eb = _bf(e)
    m1 = jnp.maximum(_dot(xg, wh1p_ref[...]) + _dot(eb, whe_ref[...])
                     + bh1_ref[...], 0.0)
    m = jnp.maximum(_dot(_bf(m1), wh2_ref[...]) + bh2_ref[...], 0.0)
    m_ref[...] = jnp.reshape(m, (EDGE_BLK * 128,))


def _edge_mlps(xg, ea128, w1p, w9, b1, w2, b2, wh1p, whe, bh1, wh2, bh2):
    nblk = E // EDGE_BLK
    full = lambda shape: pl.BlockSpec(shape, lambda i: (0,) * len(shape))
    return pl.pallas_call(
        _edge_body,
        grid=(nblk,),
        in_specs=[
            pl.BlockSpec((EDGE_BLK, 128), lambda i: (i, 0)),
            pl.BlockSpec((EDGE_BLK, 1), lambda i: (i, 0)),
            full((32, 84)), full((1, 84)), full((1, 84)),
            full((84, 39)), full((1, 39)),
            full((32, 64)), full((39, 64)), full((1, 64)),
            full((64, 128)), full((1, 128)),
        ],
        out_specs=[
            pl.BlockSpec((EDGE_BLK, 39), lambda i: (i, 0)),
            pl.BlockSpec((EDGE_BLK * 128,), lambda i: (i,)),
        ],
        out_shape=[
            jax.ShapeDtypeStruct((E, 39), jnp.float32),
            jax.ShapeDtypeStruct((E * 128,), jnp.float32),
        ],
    )(xg, ea128, w1p, w9, b1, w2, b2, wh1p, whe, bh1, wh2, bh2)


def _node_body(x_ref, s_ref, cnta_ref, cntb_ref, wp1_ref, wm1_ref, b1_ref,
               w2_ref, b2_ref, xo_ref):
    s = s_ref[...]
    cnt = jnp.maximum(cnta_ref[...][:, 0:1] + cntb_ref[...][:, 0:1], 1.0)
    mean = _bf(s / cnt)
    u1 = jnp.maximum(_dot(_bf(x_ref[...]), wp1_ref[...])
                     + _dot(mean, wm1_ref[...]) + b1_ref[...], 0.0)
    xo_ref[...] = jnp.maximum(_dot(_bf(u1), w2_ref[...]) + b2_ref[...], 0.0)


def _node_mlp(x, s, cnta, cntb, wp1, wm1, b1, w2, b2):
    nblk = N // NODE_BLK
    full = lambda shape: pl.BlockSpec(shape, lambda i: (0,) * len(shape))
    return pl.pallas_call(
        _node_body,
        grid=(nblk,),
        in_specs=[
            pl.BlockSpec((NODE_BLK, 4), lambda i: (i, 0)),
            pl.BlockSpec((NODE_BLK, 128), lambda i: (i, 0)),
            pl.BlockSpec((NODE_BLK, 32), lambda i: (i, 0)),
            pl.BlockSpec((NODE_BLK, 32), lambda i: (i, 0)),
            full((4, 256)), full((128, 256)), full((1, 256)),
            full((256, 256)), full((1, 256)),
        ],
        out_specs=pl.BlockSpec((NODE_BLK, 256), lambda i: (i, 0)),
        out_shape=jax.ShapeDtypeStruct((N, 256), jnp.float32),
    )(x, s, cnta, cntb, wp1, wm1, b1, w2, b2)


def kernel(x, edge_index, edge_attr, W_e1, b_e1, W_e2, b_e2,
           Wn1_1, bn1_1, Wn1_2, bn1_2, Wn2_1, bn2_1, Wn2_2, bn2_2):
    x16 = jnp.pad(x, ((0, 0), (0, 12)))
    idxcat2d = jnp.reshape(edge_index, (2 * E // GSUB, GSUB))
    ea128 = edge_attr
    xg = _sc_gather(x16, idxcat2d)

    bf = jnp.bfloat16
    w1p = (jnp.zeros((32, 84), jnp.float32).at[0:4].set(W_e1[0:4])
           .at[16:20].set(W_e1[4:8])).astype(bf)
    w9 = W_e1[8:9]
    wh1p = jnp.zeros((32, 64), jnp.float32).at[16:20].set(Wn1_1[0:4]).astype(bf)
    whe = Wn1_1[4:43].astype(bf)
    wp1 = Wn2_1[0:4].astype(bf)
    wm1 = Wn2_1[4:132].astype(bf)

    e, mflat = _edge_mlps(xg, ea128, w1p, w9, b_e1[None],
                          W_e2.astype(bf), b_e2[None], wh1p, whe, bn1_1[None],
                          Wn1_2.astype(bf), bn1_2[None])
    m = jnp.reshape(mflat, (E, 128))

    row2d = jnp.reshape(edge_index[0], (E // GSUB, GSUB))
    zeros = jnp.zeros((WCH, 32), jnp.float32)
    ones = jnp.ones((GSUB, 32), jnp.float32)
    s, cnta, cntb = _sc_scatter(m, row2d, zeros, ones)

    xo = _node_mlp(x, s, cnta, cntb, wp1, wm1, bn2_1[None],
                   Wn2_2.astype(bf), bn2_2[None])
    return (xo, e)


# speedup vs baseline: 6.1032x; 1.1924x over previous
import functools

import jax
import jax.numpy as jnp
from jax import lax
from jax.experimental import pallas as pl
from jax.experimental.pallas import tpu as pltpu
from jax.experimental.pallas import tpu_sc as plsc

N = 50000
E = 800000

NC = 2
NS = 16

EDGE_BLK = 6400
NODE_BLK = 2000

GK = 2000
GSUB = 125
G_NSUB = (2 * E) // (GK * NC * NS)

_sc_mesh = plsc.VectorSubcoreMesh(core_axis_name="c", subcore_axis_name="s")


GKE = 1000
NSEC = E // GK


def _gather_body(x16_ref, idxcat2d_ref, out_ref,
                 idxv0, idxv1, xsv0, xsv1, isem0, isem1, wsem0, wsem1):
    c = lax.axis_index("c")
    s = lax.axis_index("s")
    wid = s * NC + c
    nsub = GK // GSUB
    idxvs, xsvs = (idxv0, idxv1), (xsv0, xsv1)
    isems, wsems = (isem0, isem1), (wsem0, wsem1)

    def chunk_id(j):
        return wid + (NC * NS) * j

    def start_idx(j, b):
        pltpu.async_copy(idxcat2d_ref.at[pl.ds(chunk_id(j) * nsub, nsub)],
                         idxvs[b], isems[b])

    def wait_idx(b):
        pltpu.make_async_copy(idxcat2d_ref.at[pl.ds(0, nsub)],
                              idxvs[b], isems[b]).wait()

    def wait_write(b):
        pltpu.make_async_copy(xsvs[b],
                              out_ref.at[pl.ds(0, GK), pl.ds(0, 16)],
                              wsems[b]).wait()

    def work(j, b, first):
        t = chunk_id(j)
        ebase = (t % NSEC) * GK
        colo = (t // NSEC) * 16
        wait_idx(b)
        if not first:
            wait_write(b)
        for jj in range(nsub):
            pltpu.sync_copy(x16_ref.at[idxvs[b].at[jj]],
                            xsvs[b].at[pl.ds(jj * GSUB, GSUB)])
        pltpu.async_copy(xsvs[b],
                         out_ref.at[pl.ds(ebase, GK), pl.ds(colo, 16)],
                         wsems[b])

    start_idx(0, 0)
    start_idx(1, 1)
    work(0, 0, True)
    start_idx(2, 0)
    work(1, 1, True)
    start_idx(3, 1)

    def jloop(k, carry):
        j = 2 * k
        work(j, 0, False)
        start_idx(j + 2, 0)
        work(j + 1, 1, False)

        @pl.when(k < G_NSUB // 2 - 1)
        def _():
            start_idx(j + 3, 1)

        return carry

    lax.fori_loop(1, G_NSUB // 2, jloop, 0)
    work(G_NSUB - 1, 0, False)
    wait_write(0)
    wait_write(1)


def _sc_gather(x16, idxcat2d):
    return pl.kernel(
        _gather_body,
        out_type=jax.ShapeDtypeStruct((E, 128), jnp.float32),
        mesh=_sc_mesh,
        scratch_types=[
            pltpu.VMEM((GK // GSUB, GSUB), jnp.int32),
            pltpu.VMEM((GK // GSUB, GSUB), jnp.int32),
            pltpu.VMEM((GK, 16), jnp.float32),
            pltpu.VMEM((GK, 16), jnp.float32),
            pltpu.SemaphoreType.DMA,
            pltpu.SemaphoreType.DMA,
            pltpu.SemaphoreType.DMA,
            pltpu.SemaphoreType.DMA,
        ],
        compiler_params=pltpu.CompilerParams(use_tc_tiling_on_sc=False),
    )(x16, idxcat2d)


SK = 250
S_NSUB = E // (SK * NS)
C_NSUB = E // (SK * NC * NS)
ROWS_PT = N // NS
WCH = 125
SNS = SK // GSUB


def _scatter_body(m_ref, row2d_ref, zeros_ref, ones_ref,
                  s_ref, cnta_ref, cntb_ref,
                  acc, idxv0, idxv1, mrows0, mrows1, zbuf, obuf,
                  isem0, isem1, msem0, msem1):
    c = lax.axis_index("c")
    s_id = lax.axis_index("s")
    wid = s_id * NC + c
    rbase = s_id * ROWS_PT
    idxvs, mrowss = (idxv0, idxv1), (mrows0, mrows1)
    isems, msems = (isem0, isem1), (msem0, msem1)

    pltpu.sync_copy(zeros_ref, zbuf)
    pltpu.sync_copy(ones_ref, obuf)
    for k in range(ROWS_PT // WCH):
        pltpu.sync_copy(zbuf, acc.at[pl.ds(rbase + k * WCH, WCH)])
    plsc.subcore_barrier()

    for p in range(2):
        coff = (2 * c + p) * 32

        def start(j, b):
            t = s_id + NS * j
            pltpu.async_copy(row2d_ref.at[pl.ds(t * SNS, SNS)],
                             idxvs[b], isems[b])
            pltpu.async_copy(m_ref.at[pl.ds(t * SK, SK), pl.ds(coff, 32)],
                             mrowss[b], msems[b])

        def work(j, b):
            pltpu.make_async_copy(row2d_ref.at[pl.ds(0, SNS)],
                                  idxvs[b], isems[b]).wait()
            pltpu.make_async_copy(m_ref.at[pl.ds(0, SK), pl.ds(0, 32)],
                                  mrowss[b], msems[b]).wait()
            for jj in range(SNS):
                pltpu.sync_copy(mrowss[b].at[pl.ds(jj * GSUB, GSUB)],
                                acc.at[idxvs[b].at[jj]], add=True)

        start(0, 0)
        start(1, 1)

        def jloop(k, carry):
            j = 2 * k
            work(j, 0)

            @pl.when(k < S_NSUB // 2 - 1)
            def _():
                start(j + 2, 0)

            work(j + 1, 1)

            @pl.when(k < S_NSUB // 2 - 1)
            def _():
                start(j + 3, 1)

            return carry

        lax.fori_loop(0, S_NSUB // 2, jloop, 0)
        plsc.subcore_barrier()
        for k in range(ROWS_PT // WCH):
            rb = rbase + k * WCH
            wb = mrows0.at[pl.ds(0, WCH)]
            pltpu.sync_copy(acc.at[pl.ds(rb, WCH)], wb)
            pltpu.sync_copy(wb, s_ref.at[pl.ds(rb, WCH), pl.ds(coff, 32)])
            pltpu.sync_copy(zbuf, acc.at[pl.ds(rb, WCH)])
        plsc.subcore_barrier()

    def cstart(j, b):
        t2 = wid + NC * NS * j
        pltpu.async_copy(row2d_ref.at[pl.ds(t2 * SNS, SNS)],
                         idxvs[b], isems[b])

    def cwork(j, b):
        pltpu.make_async_copy(row2d_ref.at[pl.ds(0, SNS)],
                              idxvs[b], isems[b]).wait()
        for jj in range(SNS):
            pltpu.sync_copy(obuf, acc.at[idxvs[b].at[jj]], add=True)

    cstart(0, 0)
    cstart(1, 1)

    def jloop2(k, carry):
        j = 2 * k
        cwork(j, 0)

        @pl.when(k < C_NSUB // 2 - 1)
        def _():
            cstart(j + 2, 0)

        cwork(j + 1, 1)

        @pl.when(k < C_NSUB // 2 - 1)
        def _():
            cstart(j + 3, 1)

        return carry

    lax.fori_loop(0, C_NSUB // 2, jloop2, 0)
    plsc.subcore_barrier()
    for k in range(ROWS_PT // WCH):
        rb = rbase + k * WCH
        wb = mrows0.at[pl.ds(0, WCH)]
        pltpu.sync_copy(acc.at[pl.ds(rb, WCH)], wb)

        @pl.when(c == 0)
        def _():
            pltpu.sync_copy(wb, cnta_ref.at[pl.ds(rb, WCH)])

        @pl.when(c == 1)
        def _():
            pltpu.sync_copy(wb, cntb_ref.at[pl.ds(rb, WCH)])


def _sc_scatter(m, row2d, zeros, ones):
    return pl.kernel(
        _scatter_body,
        out_type=[
            jax.ShapeDtypeStruct((N, 128), jnp.float32),
            jax.ShapeDtypeStruct((N, 32), jnp.float32),
            jax.ShapeDtypeStruct((N, 32), jnp.float32),
        ],
        mesh=_sc_mesh,
        scratch_types=[
            pltpu.VMEM_SHARED((N, 32), jnp.float32),
            pltpu.VMEM((SNS, GSUB), jnp.int32),
            pltpu.VMEM((SNS, GSUB), jnp.int32),
            pltpu.VMEM((SK, 32), jnp.float32),
            pltpu.VMEM((SK, 32), jnp.float32),
            pltpu.VMEM((WCH, 32), jnp.float32),
            pltpu.VMEM((GSUB, 32), jnp.float32),
            pltpu.SemaphoreType.DMA,
            pltpu.SemaphoreType.DMA,
            pltpu.SemaphoreType.DMA,
            pltpu.SemaphoreType.DMA,
        ],
        compiler_params=pltpu.CompilerParams(use_tc_tiling_on_sc=False),
    )(m, row2d, zeros, ones)


def _dot(a, b):
    return jax.lax.dot_general(a, b, (((1,), (0,)), ((), ())),
                               preferred_element_type=jnp.float32)


def _bf(v):
    return v.astype(jnp.bfloat16)


def _edge_body(xg_ref, ea_ref, w1p_ref, w9_ref, b1_ref, w2_ref, b2_ref,
               wh1p_ref, whe_ref, bh1_ref, wh2_ref, bh2_ref,
               e_ref, m_ref):
    xg = _bf(xg_ref[...][:, 0:32])
    ea = ea_ref[...]
    e1 = jnp.maximum(_dot(xg, w1p_ref[...]) + ea * w9_ref[...] + b1_ref[...],
                     0.0)
    e = jnp.maximum(_dot(_bf(e1), w2_ref[...]) + b2_ref[...], 0.0)
    e_ref[...] = e
    eb = _bf(e)
    m1 = jnp.maximum(_dot(xg, wh1p_ref[...]) + _dot(eb, whe_ref[...])
                     + bh1_ref[...], 0.0)
    m = jnp.maximum(_dot(_bf(m1), wh2_ref[...]) + bh2_ref[...], 0.0)
    m_ref[...] = jnp.reshape(m, (EDGE_BLK * 128,))


def _edge_mlps(xg, ea128, w1p, w9, b1, w2, b2, wh1p, whe, bh1, wh2, bh2):
    nblk = E // EDGE_BLK
    full = lambda shape: pl.BlockSpec(shape, lambda i: (0,) * len(shape))
    return pl.pallas_call(
        _edge_body,
        grid=(nblk,),
        in_specs=[
            pl.BlockSpec((EDGE_BLK, 128), lambda i: (i, 0)),
            pl.BlockSpec((EDGE_BLK, 1), lambda i: (i, 0)),
            full((32, 84)), full((1, 84)), full((1, 84)),
            full((84, 39)), full((1, 39)),
            full((32, 64)), full((39, 64)), full((1, 64)),
            full((64, 128)), full((1, 128)),
        ],
        out_specs=[
            pl.BlockSpec((EDGE_BLK, 39), lambda i: (i, 0)),
            pl.BlockSpec((EDGE_BLK * 128,), lambda i: (i,)),
        ],
        out_shape=[
            jax.ShapeDtypeStruct((E, 39), jnp.float32),
            jax.ShapeDtypeStruct((E * 128,), jnp.float32),
        ],
    )(xg, ea128, w1p, w9, b1, w2, b2, wh1p, whe, bh1, wh2, bh2)


def _node_body(x_ref, s_ref, cnta_ref, cntb_ref, wp1_ref, wm1_ref, b1_ref,
               w2_ref, b2_ref, xo_ref):
    s = s_ref[...]
    cnt = jnp.maximum(cnta_ref[...][:, 0:1] + cntb_ref[...][:, 0:1], 1.0)
    mean = _bf(s / cnt)
    u1 = jnp.maximum(_dot(_bf(x_ref[...]), wp1_ref[...])
                     + _dot(mean, wm1_ref[...]) + b1_ref[...], 0.0)
    xo_ref[...] = jnp.maximum(_dot(_bf(u1), w2_ref[...]) + b2_ref[...], 0.0)


def _node_mlp(x, s, cnta, cntb, wp1, wm1, b1, w2, b2):
    nblk = N // NODE_BLK
    full = lambda shape: pl.BlockSpec(shape, lambda i: (0,) * len(shape))
    return pl.pallas_call(
        _node_body,
        grid=(nblk,),
        in_specs=[
            pl.BlockSpec((NODE_BLK, 4), lambda i: (i, 0)),
            pl.BlockSpec((NODE_BLK, 128), lambda i: (i, 0)),
            pl.BlockSpec((NODE_BLK, 32), lambda i: (i, 0)),
            pl.BlockSpec((NODE_BLK, 32), lambda i: (i, 0)),
            full((4, 256)), full((128, 256)), full((1, 256)),
            full((256, 256)), full((1, 256)),
        ],
        out_specs=pl.BlockSpec((NODE_BLK, 256), lambda i: (i, 0)),
        out_shape=jax.ShapeDtypeStruct((N, 256), jnp.float32),
    )(x, s, cnta, cntb, wp1, wm1, b1, w2, b2)


def kernel(x, edge_index, edge_attr, W_e1, b_e1, W_e2, b_e2,
           Wn1_1, bn1_1, Wn1_2, bn1_2, Wn2_1, bn2_1, Wn2_2, bn2_2):
    x16 = jnp.pad(x, ((0, 0), (0, 12)))
    idxcat2d = jnp.reshape(edge_index, (2 * E // GSUB, GSUB))
    ea128 = edge_attr
    xg = _sc_gather(x16, idxcat2d)

    bf = jnp.bfloat16
    w1p = (jnp.zeros((32, 84), jnp.float32).at[0:4].set(W_e1[0:4])
           .at[16:20].set(W_e1[4:8])).astype(bf)
    w9 = W_e1[8:9]
    wh1p = jnp.zeros((32, 64), jnp.float32).at[16:20].set(Wn1_1[0:4]).astype(bf)
    whe = Wn1_1[4:43].astype(bf)
    wp1 = Wn2_1[0:4].astype(bf)
    wm1 = Wn2_1[4:132].astype(bf)

    e, mflat = _edge_mlps(xg, ea128, w1p, w9, b_e1[None],
                          W_e2.astype(bf), b_e2[None], wh1p, whe, bn1_1[None],
                          Wn1_2.astype(bf), bn1_2[None])
    m = jnp.reshape(mflat, (E, 128))

    row2d = jnp.reshape(edge_index[0], (E // GSUB, GSUB))
    zeros = jnp.zeros((WCH, 32), jnp.float32)
    ones = jnp.ones((GSUB, 32), jnp.float32)
    s, cnta, cntb = _sc_scatter(m, row2d, zeros, ones)

    xo = _node_mlp(x, s, cnta, cntb, wp1, wm1, bn2_1[None],
                   Wn2_2.astype(bf), bn2_2[None])
    return (xo, e)


# speedup vs baseline: 6.6473x; 1.0892x over previous
import functools

import jax
import jax.numpy as jnp
from jax import lax
from jax.experimental import pallas as pl
from jax.experimental.pallas import tpu as pltpu
from jax.experimental.pallas import tpu_sc as plsc

N = 50000
E = 800000

NC = 2
NS = 16

EDGE_BLK = 6400
NODE_BLK = 2000

GK = 2000
GSUB = 125
G_NSUB = (2 * E) // (GK * NC * NS)

_sc_mesh = plsc.VectorSubcoreMesh(core_axis_name="c", subcore_axis_name="s")


GKE = 1000
NSEC = E // GK


def _gather_body(x16_ref, idxcat2d_ref, out_ref,
                 idxv0, idxv1, xsv0, xsv1, isem0, isem1, wsem0, wsem1,
                 gsem0, gsem1):
    c = lax.axis_index("c")
    s = lax.axis_index("s")
    wid = s * NC + c
    nsub = GK // GSUB
    idxvs, xsvs = (idxv0, idxv1), (xsv0, xsv1)
    isems, wsems = (isem0, isem1), (wsem0, wsem1)
    gsems = (gsem0, gsem1)

    def chunk_id(j):
        return wid + (NC * NS) * j

    def start_idx(j, b):
        pltpu.async_copy(idxcat2d_ref.at[pl.ds(chunk_id(j) * nsub, nsub)],
                         idxvs[b], isems[b])

    def wait_idx(b):
        pltpu.make_async_copy(idxcat2d_ref.at[pl.ds(0, nsub)],
                              idxvs[b], isems[b]).wait()

    def wait_write(b):
        pltpu.make_async_copy(xsvs[b],
                              out_ref.at[pl.ds(0, GK), pl.ds(0, 16)],
                              wsems[b]).wait()

    def work(j, b, first):
        t = chunk_id(j)
        ebase = (t % NSEC) * GK
        colo = (t // NSEC) * 16
        wait_idx(b)
        if not first:
            wait_write(b)
        for jj in range(nsub):
            pltpu.async_copy(x16_ref.at[idxvs[b].at[jj]],
                             xsvs[b].at[pl.ds(jj * GSUB, GSUB)], gsems[b])
        for jj in range(nsub):
            pltpu.make_async_copy(x16_ref.at[idxvs[b].at[0]],
                                  xsvs[b].at[pl.ds(jj * GSUB, GSUB)],
                                  gsems[b]).wait()
        pltpu.async_copy(xsvs[b],
                         out_ref.at[pl.ds(ebase, GK), pl.ds(colo, 16)],
                         wsems[b])

    start_idx(0, 0)
    start_idx(1, 1)
    work(0, 0, True)
    start_idx(2, 0)
    work(1, 1, True)
    start_idx(3, 1)

    def jloop(k, carry):
        j = 2 * k
        work(j, 0, False)
        start_idx(j + 2, 0)
        work(j + 1, 1, False)

        @pl.when(k < G_NSUB // 2 - 1)
        def _():
            start_idx(j + 3, 1)

        return carry

    lax.fori_loop(1, G_NSUB // 2, jloop, 0)
    work(G_NSUB - 1, 0, False)
    wait_write(0)
    wait_write(1)


def _sc_gather(x16, idxcat2d):
    return pl.kernel(
        _gather_body,
        out_type=jax.ShapeDtypeStruct((E, 128), jnp.float32),
        mesh=_sc_mesh,
        scratch_types=[
            pltpu.VMEM((GK // GSUB, GSUB), jnp.int32),
            pltpu.VMEM((GK // GSUB, GSUB), jnp.int32),
            pltpu.VMEM((GK, 16), jnp.float32),
            pltpu.VMEM((GK, 16), jnp.float32),
            pltpu.SemaphoreType.DMA,
            pltpu.SemaphoreType.DMA,
            pltpu.SemaphoreType.DMA,
            pltpu.SemaphoreType.DMA,
            pltpu.SemaphoreType.DMA,
            pltpu.SemaphoreType.DMA,
        ],
        compiler_params=pltpu.CompilerParams(use_tc_tiling_on_sc=False),
    )(x16, idxcat2d)


SK = 250
S_NSUB = E // (SK * NS)
C_NSUB = E // (SK * NC * NS)
ROWS_PT = N // NS
WCH = 125
SNS = SK // GSUB


def _scatter_body(m_ref, row2d_ref, zeros_ref, ones_ref,
                  s_ref, cnta_ref, cntb_ref,
                  acc, idxv0, idxv1, mrows0, mrows1, zbuf, obuf,
                  isem0, isem1, msem0, msem1):
    c = lax.axis_index("c")
    s_id = lax.axis_index("s")
    wid = s_id * NC + c
    rbase = s_id * ROWS_PT
    idxvs, mrowss = (idxv0, idxv1), (mrows0, mrows1)
    isems, msems = (isem0, isem1), (msem0, msem1)

    pltpu.sync_copy(zeros_ref, zbuf)
    pltpu.sync_copy(ones_ref, obuf)
    for k in range(ROWS_PT // WCH):
        pltpu.sync_copy(zbuf, acc.at[pl.ds(rbase + k * WCH, WCH)])
    plsc.subcore_barrier()

    for p in range(2):
        coff = (2 * c + p) * 32

        def start(j, b):
            t = s_id + NS * j
            pltpu.async_copy(row2d_ref.at[pl.ds(t * SNS, SNS)],
                             idxvs[b], isems[b])
            pltpu.async_copy(m_ref.at[pl.ds(t * SK, SK), pl.ds(coff, 32)],
                             mrowss[b], msems[b])

        def work(j, b):
            pltpu.make_async_copy(row2d_ref.at[pl.ds(0, SNS)],
                                  idxvs[b], isems[b]).wait()
            pltpu.make_async_copy(m_ref.at[pl.ds(0, SK), pl.ds(0, 32)],
                                  mrowss[b], msems[b]).wait()
            for jj in range(SNS):
                pltpu.sync_copy(mrowss[b].at[pl.ds(jj * GSUB, GSUB)],
                                acc.at[idxvs[b].at[jj]], add=True)

        start(0, 0)
        start(1, 1)

        def jloop(k, carry):
            j = 2 * k
            work(j, 0)

            @pl.when(k < S_NSUB // 2 - 1)
            def _():
                start(j + 2, 0)

            work(j + 1, 1)

            @pl.when(k < S_NSUB // 2 - 1)
            def _():
                start(j + 3, 1)

            return carry

        lax.fori_loop(0, S_NSUB // 2, jloop, 0)
        plsc.subcore_barrier()
        for k in range(ROWS_PT // WCH):
            rb = rbase + k * WCH
            wb = mrows0.at[pl.ds(0, WCH)]
            pltpu.sync_copy(acc.at[pl.ds(rb, WCH)], wb)
            pltpu.sync_copy(wb, s_ref.at[pl.ds(rb, WCH), pl.ds(coff, 32)])
            pltpu.sync_copy(zbuf, acc.at[pl.ds(rb, WCH)])
        plsc.subcore_barrier()

    def cstart(j, b):
        t2 = wid + NC * NS * j
        pltpu.async_copy(row2d_ref.at[pl.ds(t2 * SNS, SNS)],
                         idxvs[b], isems[b])

    def cwork(j, b):
        pltpu.make_async_copy(row2d_ref.at[pl.ds(0, SNS)],
                              idxvs[b], isems[b]).wait()
        for jj in range(SNS):
            pltpu.sync_copy(obuf, acc.at[idxvs[b].at[jj]], add=True)

    cstart(0, 0)
    cstart(1, 1)

    def jloop2(k, carry):
        j = 2 * k
        cwork(j, 0)

        @pl.when(k < C_NSUB // 2 - 1)
        def _():
            cstart(j + 2, 0)

        cwork(j + 1, 1)

        @pl.when(k < C_NSUB // 2 - 1)
        def _():
            cstart(j + 3, 1)

        return carry

    lax.fori_loop(0, C_NSUB // 2, jloop2, 0)
    plsc.subcore_barrier()
    for k in range(ROWS_PT // WCH):
        rb = rbase + k * WCH
        wb = mrows0.at[pl.ds(0, WCH)]
        pltpu.sync_copy(acc.at[pl.ds(rb, WCH)], wb)

        @pl.when(c == 0)
        def _():
            pltpu.sync_copy(wb, cnta_ref.at[pl.ds(rb, WCH)])

        @pl.when(c == 1)
        def _():
            pltpu.sync_copy(wb, cntb_ref.at[pl.ds(rb, WCH)])


def _sc_scatter(m, row2d, zeros, ones):
    return pl.kernel(
        _scatter_body,
        out_type=[
            jax.ShapeDtypeStruct((N, 128), jnp.float32),
            jax.ShapeDtypeStruct((N, 32), jnp.float32),
            jax.ShapeDtypeStruct((N, 32), jnp.float32),
        ],
        mesh=_sc_mesh,
        scratch_types=[
            pltpu.VMEM_SHARED((N, 32), jnp.float32),
            pltpu.VMEM((SNS, GSUB), jnp.int32),
            pltpu.VMEM((SNS, GSUB), jnp.int32),
            pltpu.VMEM((SK, 32), jnp.float32),
            pltpu.VMEM((SK, 32), jnp.float32),
            pltpu.VMEM((WCH, 32), jnp.float32),
            pltpu.VMEM((GSUB, 32), jnp.float32),
            pltpu.SemaphoreType.DMA,
            pltpu.SemaphoreType.DMA,
            pltpu.SemaphoreType.DMA,
            pltpu.SemaphoreType.DMA,
        ],
        compiler_params=pltpu.CompilerParams(use_tc_tiling_on_sc=False),
    )(m, row2d, zeros, ones)


def _dot(a, b):
    return jax.lax.dot_general(a, b, (((1,), (0,)), ((), ())),
                               preferred_element_type=jnp.float32)


def _bf(v):
    return v.astype(jnp.bfloat16)


def _edge_body(xg_ref, ea_ref, w1p_ref, w9_ref, b1_ref, w2_ref, b2_ref,
               wh1p_ref, whe_ref, bh1_ref, wh2_ref, bh2_ref,
               e_ref, m_ref):
    xg = _bf(xg_ref[...][:, 0:32])
    ea = ea_ref[...]
    e1 = jnp.maximum(_dot(xg, w1p_ref[...]) + ea * w9_ref[...] + b1_ref[...],
                     0.0)
    e = jnp.maximum(_dot(_bf(e1), w2_ref[...]) + b2_ref[...], 0.0)
    e_ref[...] = e
    eb = _bf(e)
    m1 = jnp.maximum(_dot(xg, wh1p_ref[...]) + _dot(eb, whe_ref[...])
                     + bh1_ref[...], 0.0)
    m = jnp.maximum(_dot(_bf(m1), wh2_ref[...]) + bh2_ref[...], 0.0)
    m_ref[...] = jnp.reshape(m, (EDGE_BLK * 128,))


def _edge_mlps(xg, ea128, w1p, w9, b1, w2, b2, wh1p, whe, bh1, wh2, bh2):
    nblk = E // EDGE_BLK
    full = lambda shape: pl.BlockSpec(shape, lambda i: (0,) * len(shape))
    return pl.pallas_call(
        _edge_body,
        grid=(nblk,),
        in_specs=[
            pl.BlockSpec((EDGE_BLK, 128), lambda i: (i, 0)),
            pl.BlockSpec((EDGE_BLK, 1), lambda i: (i, 0)),
            full((32, 84)), full((1, 84)), full((1, 84)),
            full((84, 39)), full((1, 39)),
            full((32, 64)), full((39, 64)), full((1, 64)),
            full((64, 128)), full((1, 128)),
        ],
        out_specs=[
            pl.BlockSpec((EDGE_BLK, 39), lambda i: (i, 0)),
            pl.BlockSpec((EDGE_BLK * 128,), lambda i: (i,)),
        ],
        out_shape=[
            jax.ShapeDtypeStruct((E, 39), jnp.float32),
            jax.ShapeDtypeStruct((E * 128,), jnp.float32),
        ],
    )(xg, ea128, w1p, w9, b1, w2, b2, wh1p, whe, bh1, wh2, bh2)


def _node_body(x_ref, s_ref, cnta_ref, cntb_ref, wp1_ref, wm1_ref, b1_ref,
               w2_ref, b2_ref, xo_ref):
    s = s_ref[...]
    cnt = jnp.maximum(cnta_ref[...][:, 0:1] + cntb_ref[...][:, 0:1], 1.0)
    mean = _bf(s / cnt)
    u1 = jnp.maximum(_dot(_bf(x_ref[...]), wp1_ref[...])
                     + _dot(mean, wm1_ref[...]) + b1_ref[...], 0.0)
    xo_ref[...] = jnp.maximum(_dot(_bf(u1), w2_ref[...]) + b2_ref[...], 0.0)


def _node_mlp(x, s, cnta, cntb, wp1, wm1, b1, w2, b2):
    nblk = N // NODE_BLK
    full = lambda shape: pl.BlockSpec(shape, lambda i: (0,) * len(shape))
    return pl.pallas_call(
        _node_body,
        grid=(nblk,),
        in_specs=[
            pl.BlockSpec((NODE_BLK, 4), lambda i: (i, 0)),
            pl.BlockSpec((NODE_BLK, 128), lambda i: (i, 0)),
            pl.BlockSpec((NODE_BLK, 32), lambda i: (i, 0)),
            pl.BlockSpec((NODE_BLK, 32), lambda i: (i, 0)),
            full((4, 256)), full((128, 256)), full((1, 256)),
            full((256, 256)), full((1, 256)),
        ],
        out_specs=pl.BlockSpec((NODE_BLK, 256), lambda i: (i, 0)),
        out_shape=jax.ShapeDtypeStruct((N, 256), jnp.float32),
    )(x, s, cnta, cntb, wp1, wm1, b1, w2, b2)


def kernel(x, edge_index, edge_attr, W_e1, b_e1, W_e2, b_e2,
           Wn1_1, bn1_1, Wn1_2, bn1_2, Wn2_1, bn2_1, Wn2_2, bn2_2):
    x16 = jnp.pad(x, ((0, 0), (0, 12)))
    idxcat2d = jnp.reshape(edge_index, (2 * E // GSUB, GSUB))
    ea128 = edge_attr
    xg = _sc_gather(x16, idxcat2d)

    bf = jnp.bfloat16
    w1p = (jnp.zeros((32, 84), jnp.float32).at[0:4].set(W_e1[0:4])
           .at[16:20].set(W_e1[4:8])).astype(bf)
    w9 = W_e1[8:9]
    wh1p = jnp.zeros((32, 64), jnp.float32).at[16:20].set(Wn1_1[0:4]).astype(bf)
    whe = Wn1_1[4:43].astype(bf)
    wp1 = Wn2_1[0:4].astype(bf)
    wm1 = Wn2_1[4:132].astype(bf)

    e, mflat = _edge_mlps(xg, ea128, w1p, w9, b_e1[None],
                          W_e2.astype(bf), b_e2[None], wh1p, whe, bn1_1[None],
                          Wn1_2.astype(bf), bn1_2[None])
    m = jnp.reshape(mflat, (E, 128))

    row2d = jnp.reshape(edge_index[0], (E // GSUB, GSUB))
    zeros = jnp.zeros((WCH, 32), jnp.float32)
    ones = jnp.ones((GSUB, 32), jnp.float32)
    s, cnta, cntb = _sc_scatter(m, row2d, zeros, ones)

    xo = _node_mlp(x, s, cnta, cntb, wp1, wm1, bn2_1[None],
                   Wn2_2.astype(bf), bn2_2[None])
    return (xo, e)


# speedup vs baseline: 6.6924x; 1.0068x over previous
import functools

import jax
import jax.numpy as jnp
from jax import lax
from jax.experimental import pallas as pl
from jax.experimental.pallas import tpu as pltpu
from jax.experimental.pallas import tpu_sc as plsc

N = 50000
E = 800000

NC = 2
NS = 16

EDGE_BLK = 6400
NODE_BLK = 2000

GK = 2000
GSUB = 125
G_NSUB = (2 * E) // (GK * NC * NS)

_sc_mesh = plsc.VectorSubcoreMesh(core_axis_name="c", subcore_axis_name="s")


GKE = 1000
NSEC = E // GK


def _gather_body(x16_ref, idxcat2d_ref, out_ref,
                 idxv0, idxv1, xsv0, xsv1, isem0, isem1, wsem0, wsem1,
                 gsem0, gsem1):
    c = lax.axis_index("c")
    s = lax.axis_index("s")
    wid = s * NC + c
    nsub = GK // GSUB
    idxvs, xsvs = (idxv0, idxv1), (xsv0, xsv1)
    isems, wsems = (isem0, isem1), (wsem0, wsem1)
    gsems = (gsem0, gsem1)

    def chunk_id(j):
        return wid + (NC * NS) * j

    def start_idx(j, b):
        pltpu.async_copy(idxcat2d_ref.at[pl.ds(chunk_id(j) * nsub, nsub)],
                         idxvs[b], isems[b])

    def wait_idx(b):
        pltpu.make_async_copy(idxcat2d_ref.at[pl.ds(0, nsub)],
                              idxvs[b], isems[b]).wait()

    def wait_write(b):
        pltpu.make_async_copy(xsvs[b],
                              out_ref.at[pl.ds(0, GK), pl.ds(0, 16)],
                              wsems[b]).wait()

    def work(j, b, first):
        t = chunk_id(j)
        ebase = (t % NSEC) * GK
        colo = (t // NSEC) * 16
        wait_idx(b)
        if not first:
            wait_write(b)
        for jj in range(nsub):
            pltpu.async_copy(x16_ref.at[idxvs[b].at[jj]],
                             xsvs[b].at[pl.ds(jj * GSUB, GSUB)], gsems[b])
        for jj in range(nsub):
            pltpu.make_async_copy(x16_ref.at[idxvs[b].at[0]],
                                  xsvs[b].at[pl.ds(jj * GSUB, GSUB)],
                                  gsems[b]).wait()
        pltpu.async_copy(xsvs[b],
                         out_ref.at[pl.ds(ebase, GK), pl.ds(colo, 16)],
                         wsems[b])

    start_idx(0, 0)
    start_idx(1, 1)
    work(0, 0, True)
    start_idx(2, 0)
    work(1, 1, True)
    start_idx(3, 1)

    def jloop(k, carry):
        j = 2 * k
        work(j, 0, False)
        start_idx(j + 2, 0)
        work(j + 1, 1, False)

        @pl.when(k < G_NSUB // 2 - 1)
        def _():
            start_idx(j + 3, 1)

        return carry

    lax.fori_loop(1, G_NSUB // 2, jloop, 0)
    work(G_NSUB - 1, 0, False)
    wait_write(0)
    wait_write(1)


def _sc_gather(x16, idxcat2d):
    return pl.kernel(
        _gather_body,
        out_type=jax.ShapeDtypeStruct((E, 128), jnp.float32),
        mesh=_sc_mesh,
        scratch_types=[
            pltpu.VMEM((GK // GSUB, GSUB), jnp.int32),
            pltpu.VMEM((GK // GSUB, GSUB), jnp.int32),
            pltpu.VMEM((GK, 16), jnp.float32),
            pltpu.VMEM((GK, 16), jnp.float32),
            pltpu.SemaphoreType.DMA,
            pltpu.SemaphoreType.DMA,
            pltpu.SemaphoreType.DMA,
            pltpu.SemaphoreType.DMA,
            pltpu.SemaphoreType.DMA,
            pltpu.SemaphoreType.DMA,
        ],
        compiler_params=pltpu.CompilerParams(use_tc_tiling_on_sc=False),
    )(x16, idxcat2d)


SK = 250
S_NSUB = E // (SK * NS)
C_NSUB = E // (SK * NC * NS)
ROWS_PT = N // NS
WCH = 125
SNS = SK // GSUB


def _scatter_body(m_ref, row2d_ref, zeros_ref, ones_ref,
                  s_ref, cnta_ref, cntb_ref,
                  acc, idxv0, idxv1, mrows0, mrows1, zbuf, obuf,
                  isem0, isem1, msem0, msem1, ssem0, ssem1):
    c = lax.axis_index("c")
    s_id = lax.axis_index("s")
    wid = s_id * NC + c
    rbase = s_id * ROWS_PT
    idxvs, mrowss = (idxv0, idxv1), (mrows0, mrows1)
    isems, msems = (isem0, isem1), (msem0, msem1)
    ssems = (ssem0, ssem1)

    pltpu.sync_copy(zeros_ref, zbuf)
    pltpu.sync_copy(ones_ref, obuf)
    for k in range(ROWS_PT // WCH):
        pltpu.sync_copy(zbuf, acc.at[pl.ds(rbase + k * WCH, WCH)])
    plsc.subcore_barrier()

    for p in range(2):
        coff = (2 * c + p) * 32

        def start(j, b):
            t = s_id + NS * j
            pltpu.async_copy(row2d_ref.at[pl.ds(t * SNS, SNS)],
                             idxvs[b], isems[b])
            pltpu.async_copy(m_ref.at[pl.ds(t * SK, SK), pl.ds(coff, 32)],
                             mrowss[b], msems[b])

        def work(j, b):
            pltpu.make_async_copy(row2d_ref.at[pl.ds(0, SNS)],
                                  idxvs[b], isems[b]).wait()
            pltpu.make_async_copy(m_ref.at[pl.ds(0, SK), pl.ds(0, 32)],
                                  mrowss[b], msems[b]).wait()
            for jj in range(SNS):
                pltpu.async_copy(mrowss[b].at[pl.ds(jj * GSUB, GSUB)],
                                 acc.at[idxvs[b].at[jj]], ssems[b], add=True)
            for jj in range(SNS):
                pltpu.make_async_copy(mrowss[b].at[pl.ds(jj * GSUB, GSUB)],
                                      acc.at[idxvs[b].at[jj]],
                                      ssems[b]).wait()

        start(0, 0)
        start(1, 1)

        def jloop(k, carry):
            j = 2 * k
            work(j, 0)

            @pl.when(k < S_NSUB // 2 - 1)
            def _():
                start(j + 2, 0)

            work(j + 1, 1)

            @pl.when(k < S_NSUB // 2 - 1)
            def _():
                start(j + 3, 1)

            return carry

        lax.fori_loop(0, S_NSUB // 2, jloop, 0)
        plsc.subcore_barrier()
        for k in range(ROWS_PT // WCH):
            rb = rbase + k * WCH
            wb = mrows0.at[pl.ds(0, WCH)]
            pltpu.sync_copy(acc.at[pl.ds(rb, WCH)], wb)
            pltpu.sync_copy(wb, s_ref.at[pl.ds(rb, WCH), pl.ds(coff, 32)])
            pltpu.sync_copy(zbuf, acc.at[pl.ds(rb, WCH)])
        plsc.subcore_barrier()

    def cstart(j, b):
        t2 = wid + NC * NS * j
        pltpu.async_copy(row2d_ref.at[pl.ds(t2 * SNS, SNS)],
                         idxvs[b], isems[b])

    def cwork(j, b):
        pltpu.make_async_copy(row2d_ref.at[pl.ds(0, SNS)],
                              idxvs[b], isems[b]).wait()
        for jj in range(SNS):
            pltpu.async_copy(obuf, acc.at[idxvs[b].at[jj]], ssems[b],
                             add=True)
        for jj in range(SNS):
            pltpu.make_async_copy(obuf, acc.at[idxvs[b].at[jj]],
                                  ssems[b]).wait()

    cstart(0, 0)
    cstart(1, 1)

    def jloop2(k, carry):
        j = 2 * k
        cwork(j, 0)

        @pl.when(k < C_NSUB // 2 - 1)
        def _():
            cstart(j + 2, 0)

        cwork(j + 1, 1)

        @pl.when(k < C_NSUB // 2 - 1)
        def _():
            cstart(j + 3, 1)

        return carry

    lax.fori_loop(0, C_NSUB // 2, jloop2, 0)
    plsc.subcore_barrier()
    for k in range(ROWS_PT // WCH):
        rb = rbase + k * WCH
        wb = mrows0.at[pl.ds(0, WCH)]
        pltpu.sync_copy(acc.at[pl.ds(rb, WCH)], wb)

        @pl.when(c == 0)
        def _():
            pltpu.sync_copy(wb, cnta_ref.at[pl.ds(rb, WCH)])

        @pl.when(c == 1)
        def _():
            pltpu.sync_copy(wb, cntb_ref.at[pl.ds(rb, WCH)])


def _sc_scatter(m, row2d, zeros, ones):
    return pl.kernel(
        _scatter_body,
        out_type=[
            jax.ShapeDtypeStruct((N, 128), jnp.float32),
            jax.ShapeDtypeStruct((N, 32), jnp.float32),
            jax.ShapeDtypeStruct((N, 32), jnp.float32),
        ],
        mesh=_sc_mesh,
        scratch_types=[
            pltpu.VMEM_SHARED((N, 32), jnp.float32),
            pltpu.VMEM((SNS, GSUB), jnp.int32),
            pltpu.VMEM((SNS, GSUB), jnp.int32),
            pltpu.VMEM((SK, 32), jnp.float32),
            pltpu.VMEM((SK, 32), jnp.float32),
            pltpu.VMEM((WCH, 32), jnp.float32),
            pltpu.VMEM((GSUB, 32), jnp.float32),
            pltpu.SemaphoreType.DMA,
            pltpu.SemaphoreType.DMA,
            pltpu.SemaphoreType.DMA,
            pltpu.SemaphoreType.DMA,
            pltpu.SemaphoreType.DMA,
            pltpu.SemaphoreType.DMA,
        ],
        compiler_params=pltpu.CompilerParams(use_tc_tiling_on_sc=False),
    )(m, row2d, zeros, ones)


def _dot(a, b):
    return jax.lax.dot_general(a, b, (((1,), (0,)), ((), ())),
                               preferred_element_type=jnp.float32)


def _bf(v):
    return v.astype(jnp.bfloat16)


def _edge_body(xg_ref, ea_ref, w1p_ref, w9_ref, b1_ref, w2_ref, b2_ref,
               wh1p_ref, whe_ref, bh1_ref, wh2_ref, bh2_ref,
               e_ref, m_ref):
    xg = _bf(xg_ref[...][:, 0:32])
    ea = ea_ref[...]
    e1 = jnp.maximum(_dot(xg, w1p_ref[...]) + ea * w9_ref[...] + b1_ref[...],
                     0.0)
    e = jnp.maximum(_dot(_bf(e1), w2_ref[...]) + b2_ref[...], 0.0)
    e_ref[...] = e
    eb = _bf(e)
    m1 = jnp.maximum(_dot(xg, wh1p_ref[...]) + _dot(eb, whe_ref[...])
                     + bh1_ref[...], 0.0)
    m = jnp.maximum(_dot(_bf(m1), wh2_ref[...]) + bh2_ref[...], 0.0)
    m_ref[...] = jnp.reshape(m, (EDGE_BLK * 128,))


def _edge_mlps(xg, ea128, w1p, w9, b1, w2, b2, wh1p, whe, bh1, wh2, bh2):
    nblk = E // EDGE_BLK
    full = lambda shape: pl.BlockSpec(shape, lambda i: (0,) * len(shape))
    return pl.pallas_call(
        _edge_body,
        grid=(nblk,),
        in_specs=[
            pl.BlockSpec((EDGE_BLK, 128), lambda i: (i, 0)),
            pl.BlockSpec((EDGE_BLK, 1), lambda i: (i, 0)),
            full((32, 84)), full((1, 84)), full((1, 84)),
            full((84, 39)), full((1, 39)),
            full((32, 64)), full((39, 64)), full((1, 64)),
            full((64, 128)), full((1, 128)),
        ],
        out_specs=[
            pl.BlockSpec((EDGE_BLK, 39), lambda i: (i, 0)),
            pl.BlockSpec((EDGE_BLK * 128,), lambda i: (i,)),
        ],
        out_shape=[
            jax.ShapeDtypeStruct((E, 39), jnp.float32),
            jax.ShapeDtypeStruct((E * 128,), jnp.float32),
        ],
    )(xg, ea128, w1p, w9, b1, w2, b2, wh1p, whe, bh1, wh2, bh2)


def _node_body(x_ref, s_ref, cnta_ref, cntb_ref, wp1_ref, wm1_ref, b1_ref,
               w2_ref, b2_ref, xo_ref):
    s = s_ref[...]
    cnt = jnp.maximum(cnta_ref[...][:, 0:1] + cntb_ref[...][:, 0:1], 1.0)
    mean = _bf(s / cnt)
    u1 = jnp.maximum(_dot(_bf(x_ref[...]), wp1_ref[...])
                     + _dot(mean, wm1_ref[...]) + b1_ref[...], 0.0)
    xo_ref[...] = jnp.maximum(_dot(_bf(u1), w2_ref[...]) + b2_ref[...], 0.0)


def _node_mlp(x, s, cnta, cntb, wp1, wm1, b1, w2, b2):
    nblk = N // NODE_BLK
    full = lambda shape: pl.BlockSpec(shape, lambda i: (0,) * len(shape))
    return pl.pallas_call(
        _node_body,
        grid=(nblk,),
        in_specs=[
            pl.BlockSpec((NODE_BLK, 4), lambda i: (i, 0)),
            pl.BlockSpec((NODE_BLK, 128), lambda i: (i, 0)),
            pl.BlockSpec((NODE_BLK, 32), lambda i: (i, 0)),
            pl.BlockSpec((NODE_BLK, 32), lambda i: (i, 0)),
            full((4, 256)), full((128, 256)), full((1, 256)),
            full((256, 256)), full((1, 256)),
        ],
        out_specs=pl.BlockSpec((NODE_BLK, 256), lambda i: (i, 0)),
        out_shape=jax.ShapeDtypeStruct((N, 256), jnp.float32),
    )(x, s, cnta, cntb, wp1, wm1, b1, w2, b2)


def kernel(x, edge_index, edge_attr, W_e1, b_e1, W_e2, b_e2,
           Wn1_1, bn1_1, Wn1_2, bn1_2, Wn2_1, bn2_1, Wn2_2, bn2_2):
    x16 = jnp.pad(x, ((0, 0), (0, 12)))
    idxcat2d = jnp.reshape(edge_index, (2 * E // GSUB, GSUB))
    ea128 = edge_attr
    xg = _sc_gather(x16, idxcat2d)

    bf = jnp.bfloat16
    w1p = (jnp.zeros((32, 84), jnp.float32).at[0:4].set(W_e1[0:4])
           .at[16:20].set(W_e1[4:8])).astype(bf)
    w9 = W_e1[8:9]
    wh1p = jnp.zeros((32, 64), jnp.float32).at[16:20].set(Wn1_1[0:4]).astype(bf)
    whe = Wn1_1[4:43].astype(bf)
    wp1 = Wn2_1[0:4].astype(bf)
    wm1 = Wn2_1[4:132].astype(bf)

    e, mflat = _edge_mlps(xg, ea128, w1p, w9, b_e1[None],
                          W_e2.astype(bf), b_e2[None], wh1p, whe, bn1_1[None],
                          Wn1_2.astype(bf), bn1_2[None])
    m = jnp.reshape(mflat, (E, 128))

    row2d = jnp.reshape(edge_index[0], (E // GSUB, GSUB))
    zeros = jnp.zeros((WCH, 32), jnp.float32)
    ones = jnp.ones((GSUB, 32), jnp.float32)
    s, cnta, cntb = _sc_scatter(m, row2d, zeros, ones)

    xo = _node_mlp(x, s, cnta, cntb, wp1, wm1, bn2_1[None],
                   Wn2_2.astype(bf), bn2_2[None])
    return (xo, e)
